# baseline (speedup 1.0000x reference)
import functools

import jax
import jax.numpy as jnp
from jax import lax
from jax.experimental import pallas as pl
from jax.experimental.pallas import tpu as pltpu
from jax.experimental.pallas import tpu_sc as plsc

C, L1, D = 64, 2047, 256
NUM_BINS, HID, BATCH_MAX = 32, 128, 256
L = L1 + 1
N = C * L
TWO_D = 2 * D

NC, NS = 2, 16
NW = NC * NS
ROWS_PER_W = N // NW
BLK = 128
NCHUNK = ROWS_PER_W // BLK
RING = 2
LAG = 1

TCBM = 1024
KTAB = NUM_BINS


def _expr_table_body(bins_ref, w1_ref, b1_ref, w2_ref, b2_ref, out_ref):
    nb, d = out_ref.shape
    bins = bins_ref[...]
    vals = lax.broadcasted_iota(jnp.int32, (nb, 1), 0).astype(jnp.float32)
    h = jnp.maximum(vals * w1_ref[...] + b1_ref[...], 0.0)
    enc = lax.dot_general(h, w2_ref[...], (((1,), (0,)), ((), ())),
                          precision=lax.Precision.DEFAULT) + b2_ref[...]
    sim = lax.dot_general(enc, bins, (((1,), (1,)), ((), ())),
                          precision=lax.Precision.DEFAULT)
    col = lax.broadcasted_iota(jnp.int32, (nb, nb), 1)
    sim = jnp.where(col == 0, -1e30, sim)
    m = jnp.max(sim, axis=-1, keepdims=True)
    p = jnp.exp(sim - m)
    p = p / jnp.sum(p, axis=-1, keepdims=True)
    emb = lax.dot_general(p, bins, (((1,), (0,)), ((), ())),
                          precision=lax.Precision.DEFAULT)
    row = lax.broadcasted_iota(jnp.int32, (nb, d), 0)
    out_ref[...] = jnp.where(row == 0, bins[0:1, :], emb)


def _expr_table(bin_embeddings, w1, b1, w2, b2):
    return pl.pallas_call(
        _expr_table_body,
        out_shape=jax.ShapeDtypeStruct((NUM_BINS, D), jnp.float32),
    )(bin_embeddings, w1.reshape(1, HID), b1.reshape(1, HID),
      w2, b2.reshape(1, D))


def _onehot_rows(e, ct, nrow, ktab):
    k = lax.broadcasted_iota(jnp.int32, (1, ktab), 1)
    oh = (e.reshape(nrow, 1) == k).astype(jnp.bfloat16)
    t1 = ct.astype(jnp.bfloat16)
    r1 = ct - t1.astype(jnp.float32)
    t2 = r1.astype(jnp.bfloat16)
    t3 = (r1 - t2.astype(jnp.float32)).astype(jnp.bfloat16)
    dims = (((1,), (0,)), ((), ()))
    acc = lax.dot_general(oh, t1, dims, preferred_element_type=jnp.float32)
    acc = acc + lax.dot_general(oh, t2, dims,
                                preferred_element_type=jnp.float32)
    acc = acc + lax.dot_general(oh, t3, dims,
                                preferred_element_type=jnp.float32)
    return acc


def _expr_half_body(out_any, eidx_ref, ct_ref, out_ref):
    del out_any
    out_ref[...] = _onehot_rows(eidx_ref[0, 0, :], ct_ref[...], TCBM, KTAB)


def _expr_half(sc_out, eidx2, ctable):
    return pl.pallas_call(
        _expr_half_body,
        grid=(N // TCBM,),
        in_specs=[
            pl.BlockSpec(memory_space=pl.ANY),
            pl.BlockSpec((1, 1, TCBM), lambda j: (j, 0, 0)),
            pl.BlockSpec((KTAB, D), lambda j: (0, 0)),
        ],
        out_specs=pl.BlockSpec((TCBM, D), lambda j: (j, 1)),
        out_shape=jax.ShapeDtypeStruct((N, TWO_D), jnp.float32),
        input_output_aliases={0: 0},
    )(sc_out, eidx2, ctable)


def _row0_batch_body(out_any, bidx_ref, bt_ref, out_ref):
    del out_any
    out_ref[...] = _onehot_rows(bidx_ref[0, 0, :], bt_ref[...], 8, BATCH_MAX)


def _row0_batch(out_flat, bidx, batch_table):
    return pl.pallas_call(
        _row0_batch_body,
        grid=(C // 8,),
        in_specs=[
            pl.BlockSpec(memory_space=pl.ANY),
            pl.BlockSpec((1, 1, 8), lambda j: (j, 0, 0)),
            pl.BlockSpec((BATCH_MAX, D), lambda j: (0, 0)),
        ],
        out_specs=pl.BlockSpec((8, D), lambda j: (j, 1)),
        out_shape=jax.ShapeDtypeStruct((C, L * TWO_D), jnp.float32),
        input_output_aliases={0: 0},
    )(out_flat, bidx, batch_table)


def _sc_body(gidx_hbm, cidx_hbm, gene_t, cond_t, out_hbm, *scr):
    gidx_v = scr[0]
    obuf = scr[1:1 + RING]
    ci_v, cbuf, fbuf = scr[1 + RING:4 + RING]
    sem_g = scr[4 + RING:4 + 2 * RING]
    sem_w = scr[4 + 2 * RING:4 + 3 * RING]

    wid = lax.axis_index("s") * NC + lax.axis_index("c")
    base_w = pl.multiple_of(wid * ROWS_PER_W, ROWS_PER_W)
    chunk0 = pl.multiple_of(wid * NCHUNK, NCHUNK)

    pltpu.sync_copy(gidx_hbm.at[pl.ds(chunk0, NCHUNK)], gidx_v)

    def issue_gathers(i, p):
        pltpu.async_copy(gene_t.at[gidx_v.at[i]], obuf[p], sem_g[p])

    def wait_gathers(i, p):
        pltpu.make_async_copy(gene_t.at[gidx_v.at[i]], obuf[p],
                              sem_g[p]).wait()

    def issue_write(i, p):
        base = pl.multiple_of(base_w + i * BLK, BLK)
        pltpu.async_copy(obuf[p], out_hbm.at[pl.ds(base, BLK), pl.ds(0, D)],
                         sem_w[p])

    def wait_write(p):
        pltpu.make_async_copy(obuf[p],
                              out_hbm.at[pl.ds(base_w, BLK), pl.ds(0, D)],
                              sem_w[p]).wait()

    for i in range(LAG):
        issue_gathers(i, i)

    def step(i, p):
        pb = (p + LAG) % RING

        @pl.when(i + LAG < NCHUNK)
        def _():
            @pl.when(i + LAG >= RING)
            def _():
                wait_write(pb)
            issue_gathers(i + LAG, pb)

        wait_gathers(i, p)
        issue_write(i, p)

    def group(j, carry):
        for u in range(RING):
            step(RING * j + u, u)
        return carry

    lax.fori_loop(0, NCHUNK // RING, group, 0)
    for w in range(NCHUNK - RING + LAG, NCHUNK):
        wait_write(w % RING)

    c0 = 2 * wid
    pltpu.sync_copy(cidx_hbm.at[pl.ds(pl.multiple_of(c0 * 4, 8), 8)], ci_v)
    pltpu.async_copy(cond_t.at[ci_v], cbuf, sem_g[0]).wait()
    for cell in range(2):
        row = (c0 + cell) * L
        for j in range(D // 16):
            p = j * 16
            fbuf[0, pl.ds(p, 16)] = cbuf[4 * cell + p // 64, pl.ds(p % 64, 16)]
        pltpu.sync_copy(fbuf, out_hbm.at[pl.ds(row, 1), pl.ds(0, D)])


@functools.partial(jax.jit, static_argnames=())
def _sc_assemble(gidx, cidx, gene_table, cond_table):
    mesh = plsc.VectorSubcoreMesh(core_axis_name="c", subcore_axis_name="s")
    return pl.kernel(
        _sc_body,
        out_type=jax.ShapeDtypeStruct((N, TWO_D), jnp.float32),
        mesh=mesh,
        scratch_types=(
            [pltpu.VMEM((NCHUNK, BLK), jnp.int32)]
            + [pltpu.VMEM((BLK, D), jnp.float32)] * RING
            + [
                pltpu.VMEM((8,), jnp.int32),
                pltpu.VMEM((8, 128), jnp.float32),
                pltpu.VMEM((1, D), jnp.float32),
            ]
            + [pltpu.SemaphoreType.DMA] * (2 * RING)
        ),
    )(gidx, cidx, gene_table, cond_table)


def kernel(gene, expr, cond, batch, pad, gene_table, bin_embeddings,
           W1, b1, W2, b2, cond_table, batch_table):
    expr_table = _expr_table(bin_embeddings, W1, b1, W2, b2)

    zcol = jnp.zeros((C, 1), jnp.int32)
    gidx = jnp.concatenate([zcol, gene], axis=1).reshape(N // BLK, BLK)
    cidx = cond.reshape(-1)

    eidx2 = jnp.concatenate([zcol, expr], axis=1).reshape(N // TCBM, 1, TCBM)
    bidx = batch.reshape(C // 8, 1, 8)

    cond_table_p = jnp.pad(cond_table, ((0, 0), (0, 64)))

    sc_out = _sc_assemble(gidx, cidx, gene_table, cond_table_p)
    out = _expr_half(sc_out, eidx2, expr_table)
    out = _row0_batch(out.reshape(C, L * TWO_D), bidx, batch_table)

    final_emb = out.reshape(C, L, TWO_D)
    key_padding_mask = jnp.concatenate(
        [jnp.zeros((C, 1), dtype=bool), pad.astype(bool)], axis=1)
    return (final_emb, key_padding_mask)


# speedup vs baseline: 3.1738x; 3.1738x over previous
import functools

import jax
import jax.numpy as jnp
from jax import lax
from jax.experimental import pallas as pl
from jax.experimental.pallas import tpu as pltpu
from jax.experimental.pallas import tpu_sc as plsc

C, L1, D = 64, 2047, 256
NUM_BINS, HID, BATCH_MAX = 32, 128, 256
L = L1 + 1
N = C * L
TWO_D = 2 * D

NC, NS = 2, 16
NW = NC * NS
ROWS_PER_W = N // NW
BLK = 128
NCHUNK = ROWS_PER_W // BLK
RING = 2
LAG = 1

TCBM = 1024
KTAB = NUM_BINS


def _expr_table_body(bins_ref, w1_ref, b1_ref, w2_ref, b2_ref, out_ref):
    nb, d = out_ref.shape
    bins = bins_ref[...]
    vals = lax.broadcasted_iota(jnp.int32, (nb, 1), 0).astype(jnp.float32)
    h = jnp.maximum(vals * w1_ref[...] + b1_ref[...], 0.0)
    enc = lax.dot_general(h, w2_ref[...], (((1,), (0,)), ((), ())),
                          precision=lax.Precision.DEFAULT) + b2_ref[...]
    sim = lax.dot_general(enc, bins, (((1,), (1,)), ((), ())),
                          precision=lax.Precision.DEFAULT)
    col = lax.broadcasted_iota(jnp.int32, (nb, nb), 1)
    sim = jnp.where(col == 0, -1e30, sim)
    m = jnp.max(sim, axis=-1, keepdims=True)
    p = jnp.exp(sim - m)
    p = p / jnp.sum(p, axis=-1, keepdims=True)
    emb = lax.dot_general(p, bins, (((1,), (0,)), ((), ())),
                          precision=lax.Precision.DEFAULT)
    row = lax.broadcasted_iota(jnp.int32, (nb, d), 0)
    out_ref[...] = jnp.where(row == 0, bins[0:1, :], emb)


def _expr_table(bin_embeddings, w1, b1, w2, b2):
    return pl.pallas_call(
        _expr_table_body,
        out_shape=jax.ShapeDtypeStruct((NUM_BINS, D), jnp.float32),
    )(bin_embeddings, w1.reshape(1, HID), b1.reshape(1, HID),
      w2, b2.reshape(1, D))


def _onehot_rows(e, ct, nrow, ktab):
    k = lax.broadcasted_iota(jnp.int32, (1, ktab), 1)
    oh = (e.reshape(nrow, 1) == k).astype(jnp.bfloat16)
    t1 = ct.astype(jnp.bfloat16)
    r1 = ct - t1.astype(jnp.float32)
    t2 = r1.astype(jnp.bfloat16)
    t3 = (r1 - t2.astype(jnp.float32)).astype(jnp.bfloat16)
    dims = (((1,), (0,)), ((), ()))
    acc = lax.dot_general(oh, t1, dims, preferred_element_type=jnp.float32)
    acc = acc + lax.dot_general(oh, t2, dims,
                                preferred_element_type=jnp.float32)
    acc = acc + lax.dot_general(oh, t3, dims,
                                preferred_element_type=jnp.float32)
    return acc


def _expr_half_body(out_any, eidx_ref, bidx_ref, ct_ref, bt_ref, out_ref):
    del out_any
    out_ref[...] = _onehot_rows(eidx_ref[0, 0, :], ct_ref[...], TCBM, KTAB)

    @pl.when(pl.program_id(0) % 2 == 0)
    def _():
        b = bidx_ref[0, 0, 0]
        m = lax.broadcasted_iota(jnp.int32, (BATCH_MAX, 1), 0) == b
        row = jnp.sum(jnp.where(m, bt_ref[...], 0.0), axis=0, keepdims=True)
        out_ref[pl.ds(0, 1), :] = row


def _expr_half(sc_out, eidx2, bidx_blk, ctable, batch_table):
    return pl.pallas_call(
        _expr_half_body,
        grid=(N // TCBM,),
        in_specs=[
            pl.BlockSpec(memory_space=pl.ANY),
            pl.BlockSpec((1, 1, TCBM), lambda j: (j, 0, 0)),
            pl.BlockSpec((1, 1, 8), lambda j: (j, 0, 0)),
            pl.BlockSpec((KTAB, D), lambda j: (0, 0)),
            pl.BlockSpec((BATCH_MAX, D), lambda j: (0, 0)),
        ],
        out_specs=pl.BlockSpec((TCBM, D), lambda j: (j, 1)),
        out_shape=jax.ShapeDtypeStruct((N, TWO_D), jnp.float32),
        input_output_aliases={0: 0},
    )(sc_out, eidx2, bidx_blk, ctable, batch_table)


def _sc_body(gidx_hbm, cidx_hbm, gene_t, cond_t, out_hbm, *scr):
    gidx_v = scr[0]
    obuf = scr[1:1 + RING]
    ci_v, cbuf, fbuf = scr[1 + RING:4 + RING]
    sem_g = scr[4 + RING:4 + 2 * RING]
    sem_w = scr[4 + 2 * RING:4 + 3 * RING]

    wid = lax.axis_index("s") * NC + lax.axis_index("c")
    base_w = pl.multiple_of(wid * ROWS_PER_W, ROWS_PER_W)
    chunk0 = pl.multiple_of(wid * NCHUNK, NCHUNK)

    pltpu.sync_copy(gidx_hbm.at[pl.ds(chunk0, NCHUNK)], gidx_v)

    def issue_gathers(i, p):
        pltpu.async_copy(gene_t.at[gidx_v.at[i]], obuf[p], sem_g[p])

    def wait_gathers(i, p):
        pltpu.make_async_copy(gene_t.at[gidx_v.at[i]], obuf[p],
                              sem_g[p]).wait()

    def issue_write(i, p):
        base = pl.multiple_of(base_w + i * BLK, BLK)
        pltpu.async_copy(obuf[p], out_hbm.at[pl.ds(base, BLK), pl.ds(0, D)],
                         sem_w[p])

    def wait_write(p):
        pltpu.make_async_copy(obuf[p],
                              out_hbm.at[pl.ds(base_w, BLK), pl.ds(0, D)],
                              sem_w[p]).wait()

    for i in range(LAG):
        issue_gathers(i, i)

    def step(i, p):
        pb = (p + LAG) % RING

        @pl.when(i + LAG < NCHUNK)
        def _():
            @pl.when(i + LAG >= RING)
            def _():
                wait_write(pb)
            issue_gathers(i + LAG, pb)

        wait_gathers(i, p)
        issue_write(i, p)

    def group(j, carry):
        for u in range(RING):
            step(RING * j + u, u)
        return carry

    lax.fori_loop(0, NCHUNK // RING, group, 0)
    for w in range(NCHUNK - RING + LAG, NCHUNK):
        wait_write(w % RING)

    c0 = 2 * wid
    pltpu.sync_copy(cidx_hbm.at[pl.ds(pl.multiple_of(c0 * 4, 8), 8)], ci_v)
    pltpu.async_copy(cond_t.at[ci_v], cbuf, sem_g[0]).wait()
    for cell in range(2):
        row = (c0 + cell) * L
        for j in range(D // 16):
            p = j * 16
            fbuf[0, pl.ds(p, 16)] = cbuf[4 * cell + p // 64, pl.ds(p % 64, 16)]
        pltpu.sync_copy(fbuf, out_hbm.at[pl.ds(row, 1), pl.ds(0, D)])


@functools.partial(jax.jit, static_argnames=())
def _sc_assemble(gidx, cidx, gene_table, cond_table):
    mesh = plsc.VectorSubcoreMesh(core_axis_name="c", subcore_axis_name="s")
    return pl.kernel(
        _sc_body,
        out_type=jax.ShapeDtypeStruct((N, TWO_D), jnp.float32),
        mesh=mesh,
        scratch_types=(
            [pltpu.VMEM((NCHUNK, BLK), jnp.int32)]
            + [pltpu.VMEM((BLK, D), jnp.float32)] * RING
            + [
                pltpu.VMEM((8,), jnp.int32),
                pltpu.VMEM((8, 128), jnp.float32),
                pltpu.VMEM((1, D), jnp.float32),
            ]
            + [pltpu.SemaphoreType.DMA] * (2 * RING)
        ),
    )(gidx, cidx, gene_table, cond_table)


def kernel(gene, expr, cond, batch, pad, gene_table, bin_embeddings,
           W1, b1, W2, b2, cond_table, batch_table):
    expr_table = _expr_table(bin_embeddings, W1, b1, W2, b2)

    zcol = jnp.zeros((C, 1), jnp.int32)
    gidx = jnp.concatenate([zcol, gene], axis=1).reshape(N // BLK, BLK)
    cidx = cond.reshape(-1)

    eidx2 = jnp.concatenate([zcol, expr], axis=1).reshape(N // TCBM, 1, TCBM)
    bidx_blk = jnp.broadcast_to(
        jnp.stack([batch[:, 0], jnp.zeros((C,), jnp.int32)],
                  axis=1).reshape(N // TCBM, 1, 1),
        (N // TCBM, 1, 8)).copy()

    cond_table_p = jnp.pad(cond_table, ((0, 0), (0, 64)))

    sc_out = _sc_assemble(gidx, cidx, gene_table, cond_table_p)
    out = _expr_half(sc_out, eidx2, bidx_blk, expr_table, batch_table)

    final_emb = out.reshape(C, L, TWO_D)
    key_padding_mask = jnp.concatenate(
        [jnp.zeros((C, 1), dtype=bool), pad.astype(bool)], axis=1)
    return (final_emb, key_padding_mask)


# speedup vs baseline: 3.6960x; 1.1645x over previous
import functools

import jax
import jax.numpy as jnp
from jax import lax
from jax.experimental import pallas as pl
from jax.experimental.pallas import tpu as pltpu
from jax.experimental.pallas import tpu_sc as plsc

C, L1, D = 64, 2047, 256
NUM_BINS, HID, BATCH_MAX = 32, 128, 256
L = L1 + 1
N = C * L
TWO_D = 2 * D

NC, NS = 2, 16
NW = NC * NS
ROWS_PER_W = N // NW
BLK = 128
NCHUNK = ROWS_PER_W // BLK
RING = 2
LAG = 1

TCBM = 2048
KTAB = NUM_BINS


def _expr_table_body(bins_ref, w1_ref, b1_ref, w2_ref, b2_ref, out_ref):
    nb, d = out_ref.shape
    bins = bins_ref[...]
    vals = lax.broadcasted_iota(jnp.int32, (nb, 1), 0).astype(jnp.float32)
    h = jnp.maximum(vals * w1_ref[...] + b1_ref[...], 0.0)
    enc = lax.dot_general(h, w2_ref[...], (((1,), (0,)), ((), ())),
                          precision=lax.Precision.DEFAULT) + b2_ref[...]
    sim = lax.dot_general(enc, bins, (((1,), (1,)), ((), ())),
                          precision=lax.Precision.DEFAULT)
    col = lax.broadcasted_iota(jnp.int32, (nb, nb), 1)
    sim = jnp.where(col == 0, -1e30, sim)
    m = jnp.max(sim, axis=-1, keepdims=True)
    p = jnp.exp(sim - m)
    p = p / jnp.sum(p, axis=-1, keepdims=True)
    emb = lax.dot_general(p, bins, (((1,), (0,)), ((), ())),
                          precision=lax.Precision.DEFAULT)
    row = lax.broadcasted_iota(jnp.int32, (nb, d), 0)
    out_ref[...] = jnp.where(row == 0, bins[0:1, :], emb)


def _expr_table(bin_embeddings, w1, b1, w2, b2):
    return pl.pallas_call(
        _expr_table_body,
        out_shape=jax.ShapeDtypeStruct((NUM_BINS, D), jnp.float32),
    )(bin_embeddings, w1.reshape(1, HID), b1.reshape(1, HID),
      w2, b2.reshape(1, D))


def _onehot_rows(e, ct, nrow, ktab):
    k = lax.broadcasted_iota(jnp.int32, (1, ktab), 1)
    oh = (e.reshape(nrow, 1) == k).astype(jnp.bfloat16)
    t1 = ct.astype(jnp.bfloat16)
    r1 = ct - t1.astype(jnp.float32)
    t2 = r1.astype(jnp.bfloat16)
    t3 = (r1 - t2.astype(jnp.float32)).astype(jnp.bfloat16)
    dims = (((1,), (0,)), ((), ()))
    acc = lax.dot_general(oh, t1, dims, preferred_element_type=jnp.float32)
    acc = acc + lax.dot_general(oh, t2, dims,
                                preferred_element_type=jnp.float32)
    acc = acc + lax.dot_general(oh, t3, dims,
                                preferred_element_type=jnp.float32)
    return acc


def _expr_half_body(out_any, eidx_ref, bidx_ref, ct_ref, bt_ref, out_ref):
    del out_any
    out_ref[...] = _onehot_rows(eidx_ref[0, 0, :], ct_ref[...], TCBM, KTAB)

    b = bidx_ref[0, 0, 0]
    m = lax.broadcasted_iota(jnp.int32, (BATCH_MAX, 1), 0) == b
    row = jnp.sum(jnp.where(m, bt_ref[...], 0.0), axis=0, keepdims=True)
    out_ref[pl.ds(0, 1), :] = row


def _expr_half(sc_out, eidx2, bidx_blk, ctable, batch_table):
    return pl.pallas_call(
        _expr_half_body,
        grid=(N // TCBM,),
        in_specs=[
            pl.BlockSpec(memory_space=pl.ANY),
            pl.BlockSpec((1, 1, TCBM), lambda j: (j, 0, 0)),
            pl.BlockSpec((1, 1, 8), lambda j: (j, 0, 0)),
            pl.BlockSpec((KTAB, D), lambda j: (0, 0)),
            pl.BlockSpec((BATCH_MAX, D), lambda j: (0, 0)),
        ],
        out_specs=pl.BlockSpec((TCBM, D), lambda j: (j, 1)),
        out_shape=jax.ShapeDtypeStruct((N, TWO_D), jnp.float32),
        input_output_aliases={0: 0},
    )(sc_out, eidx2, bidx_blk, ctable, batch_table)


def _sc_body(gidx_hbm, cidx_hbm, gene_t, cond_t, out_hbm, *scr):
    gidx_v = scr[0]
    obuf = scr[1:1 + RING]
    ci_v, cbuf, fbuf = scr[1 + RING:4 + RING]
    sem_g = scr[4 + RING:4 + 2 * RING]
    sem_w = scr[4 + 2 * RING:4 + 3 * RING]

    wid = lax.axis_index("s") * NC + lax.axis_index("c")
    base_w = pl.multiple_of(wid * ROWS_PER_W, ROWS_PER_W)
    chunk0 = pl.multiple_of(wid * NCHUNK, NCHUNK)

    pltpu.sync_copy(gidx_hbm.at[pl.ds(chunk0, NCHUNK)], gidx_v)

    def issue_gathers(i, p):
        pltpu.async_copy(gene_t.at[gidx_v.at[i]], obuf[p], sem_g[p])

    def wait_gathers(i, p):
        pltpu.make_async_copy(gene_t.at[gidx_v.at[i]], obuf[p],
                              sem_g[p]).wait()

    def issue_write(i, p):
        base = pl.multiple_of(base_w + i * BLK, BLK)
        pltpu.async_copy(obuf[p], out_hbm.at[pl.ds(base, BLK), pl.ds(0, D)],
                         sem_w[p])

    def wait_write(p):
        pltpu.make_async_copy(obuf[p],
                              out_hbm.at[pl.ds(base_w, BLK), pl.ds(0, D)],
                              sem_w[p]).wait()

    for i in range(LAG):
        issue_gathers(i, i)

    def step(i, p):
        pb = (p + LAG) % RING

        @pl.when(i + LAG < NCHUNK)
        def _():
            @pl.when(i + LAG >= RING)
            def _():
                wait_write(pb)
            issue_gathers(i + LAG, pb)

        wait_gathers(i, p)
        issue_write(i, p)

    def group(j, carry):
        for u in range(RING):
            step(RING * j + u, u)
        return carry

    lax.fori_loop(0, NCHUNK // RING, group, 0)
    for w in range(NCHUNK - RING + LAG, NCHUNK):
        wait_write(w % RING)

    c0 = 2 * wid
    pltpu.sync_copy(cidx_hbm.at[pl.ds(pl.multiple_of(c0 * 4, 8), 8)], ci_v)
    pltpu.async_copy(cond_t.at[ci_v], cbuf, sem_g[0]).wait()
    for cell in range(2):
        row = (c0 + cell) * L
        for j in range(D // 16):
            p = j * 16
            fbuf[0, pl.ds(p, 16)] = cbuf[4 * cell + p // 64, pl.ds(p % 64, 16)]
        pltpu.sync_copy(fbuf, out_hbm.at[pl.ds(row, 1), pl.ds(0, D)])


@functools.partial(jax.jit, static_argnames=())
def _sc_assemble(gidx, cidx, gene_table, cond_table):
    mesh = plsc.VectorSubcoreMesh(core_axis_name="c", subcore_axis_name="s")
    return pl.kernel(
        _sc_body,
        out_type=jax.ShapeDtypeStruct((N, TWO_D), jnp.float32),
        mesh=mesh,
        scratch_types=(
            [pltpu.VMEM((NCHUNK, BLK), jnp.int32)]
            + [pltpu.VMEM((BLK, D), jnp.float32)] * RING
            + [
                pltpu.VMEM((8,), jnp.int32),
                pltpu.VMEM((8, 128), jnp.float32),
                pltpu.VMEM((1, D), jnp.float32),
            ]
            + [pltpu.SemaphoreType.DMA] * (2 * RING)
        ),
    )(gidx, cidx, gene_table, cond_table)


def kernel(gene, expr, cond, batch, pad, gene_table, bin_embeddings,
           W1, b1, W2, b2, cond_table, batch_table):
    expr_table = _expr_table(bin_embeddings, W1, b1, W2, b2)

    zcol = jnp.zeros((C, 1), jnp.int32)
    gidx = jnp.concatenate([zcol, gene], axis=1).reshape(N // BLK, BLK)
    cidx = cond.reshape(-1)

    eidx2 = jnp.concatenate([zcol, expr], axis=1).reshape(N // TCBM, 1, TCBM)
    bidx_blk = jnp.broadcast_to(batch.reshape(C, 1, 1), (C, 1, 8)).copy()

    cond_table_p = jnp.pad(cond_table, ((0, 0), (0, 64)))

    sc_out = _sc_assemble(gidx, cidx, gene_table, cond_table_p)
    out = _expr_half(sc_out, eidx2, bidx_blk, expr_table, batch_table)

    final_emb = out.reshape(C, L, TWO_D)
    key_padding_mask = jnp.concatenate(
        [jnp.zeros((C, 1), dtype=bool), pad.astype(bool)], axis=1)
    return (final_emb, key_padding_mask)


# speedup vs baseline: 4.0252x; 1.0891x over previous
import functools

import jax
import jax.numpy as jnp
from jax import lax
from jax.experimental import pallas as pl
from jax.experimental.pallas import tpu as pltpu
from jax.experimental.pallas import tpu_sc as plsc

C, L1, D = 64, 2047, 256
NUM_BINS, HID, BATCH_MAX = 32, 128, 256
L = L1 + 1
N = C * L
TWO_D = 2 * D

NC, NS = 2, 16
NW = NC * NS
ROWS_PER_W = N // NW
BLK = 128
NCHUNK = ROWS_PER_W // BLK
RING = 2
LAG = 1

TCBM = 4096
KTAB = NUM_BINS


def _expr_table_body(bins_ref, w1_ref, b1_ref, w2_ref, b2_ref, out_ref):
    nb, d = out_ref.shape
    bins = bins_ref[...]
    vals = lax.broadcasted_iota(jnp.int32, (nb, 1), 0).astype(jnp.float32)
    h = jnp.maximum(vals * w1_ref[...] + b1_ref[...], 0.0)
    enc = lax.dot_general(h, w2_ref[...], (((1,), (0,)), ((), ())),
                          precision=lax.Precision.DEFAULT) + b2_ref[...]
    sim = lax.dot_general(enc, bins, (((1,), (1,)), ((), ())),
                          precision=lax.Precision.DEFAULT)
    col = lax.broadcasted_iota(jnp.int32, (nb, nb), 1)
    sim = jnp.where(col == 0, -1e30, sim)
    m = jnp.max(sim, axis=-1, keepdims=True)
    p = jnp.exp(sim - m)
    p = p / jnp.sum(p, axis=-1, keepdims=True)
    emb = lax.dot_general(p, bins, (((1,), (0,)), ((), ())),
                          precision=lax.Precision.DEFAULT)
    row = lax.broadcasted_iota(jnp.int32, (nb, d), 0)
    out_ref[...] = jnp.where(row == 0, bins[0:1, :], emb)


def _expr_table(bin_embeddings, w1, b1, w2, b2):
    return pl.pallas_call(
        _expr_table_body,
        out_shape=jax.ShapeDtypeStruct((NUM_BINS, D), jnp.float32),
    )(bin_embeddings, w1.reshape(1, HID), b1.reshape(1, HID),
      w2, b2.reshape(1, D))


def _onehot_rows(e, ct, nrow, ktab):
    k = lax.broadcasted_iota(jnp.int32, (1, ktab), 1)
    oh = (e.reshape(nrow, 1) == k).astype(jnp.bfloat16)
    t1 = ct.astype(jnp.bfloat16)
    r1 = ct - t1.astype(jnp.float32)
    t2 = r1.astype(jnp.bfloat16)
    t3 = (r1 - t2.astype(jnp.float32)).astype(jnp.bfloat16)
    dims = (((1,), (0,)), ((), ()))
    acc = lax.dot_general(oh, t1, dims, preferred_element_type=jnp.float32)
    acc = acc + lax.dot_general(oh, t2, dims,
                                preferred_element_type=jnp.float32)
    acc = acc + lax.dot_general(oh, t3, dims,
                                preferred_element_type=jnp.float32)
    return acc


def _expr_half_body(out_any, eidx_ref, bidx_ref, ct_ref, bt_ref, out_ref):
    del out_any
    out_ref[...] = _onehot_rows(eidx_ref[0, 0, :], ct_ref[...], TCBM, KTAB)

    for cell in range(TCBM // L):
        b = bidx_ref[0, 0, cell]
        m = lax.broadcasted_iota(jnp.int32, (BATCH_MAX, 1), 0) == b
        row = jnp.sum(jnp.where(m, bt_ref[...], 0.0), axis=0, keepdims=True)
        out_ref[pl.ds(cell * L, 1), :] = row


def _expr_half(sc_out, eidx2, bidx_blk, ctable, batch_table):
    return pl.pallas_call(
        _expr_half_body,
        grid=(N // TCBM,),
        in_specs=[
            pl.BlockSpec(memory_space=pl.ANY),
            pl.BlockSpec((1, 1, TCBM), lambda j: (j, 0, 0)),
            pl.BlockSpec((1, 1, 8), lambda j: (j, 0, 0)),
            pl.BlockSpec((KTAB, D), lambda j: (0, 0)),
            pl.BlockSpec((BATCH_MAX, D), lambda j: (0, 0)),
        ],
        out_specs=pl.BlockSpec((TCBM, D), lambda j: (j, 1)),
        out_shape=jax.ShapeDtypeStruct((N, TWO_D), jnp.float32),
        input_output_aliases={0: 0},
    )(sc_out, eidx2, bidx_blk, ctable, batch_table)


def _sc_body(gidx_hbm, cidx_hbm, gene_t, cond_t, out_hbm, *scr):
    gidx_v = scr[0]
    obuf = scr[1:1 + RING]
    ci_v, cbuf, fbuf = scr[1 + RING:4 + RING]
    sem_g = scr[4 + RING:4 + 2 * RING]
    sem_w = scr[4 + 2 * RING:4 + 3 * RING]

    wid = lax.axis_index("s") * NC + lax.axis_index("c")
    base_w = pl.multiple_of(wid * ROWS_PER_W, ROWS_PER_W)
    chunk0 = pl.multiple_of(wid * NCHUNK, NCHUNK)

    pltpu.sync_copy(gidx_hbm.at[pl.ds(chunk0, NCHUNK)], gidx_v)

    def issue_gathers(i, p):
        pltpu.async_copy(gene_t.at[gidx_v.at[i]], obuf[p], sem_g[p])

    def wait_gathers(i, p):
        pltpu.make_async_copy(gene_t.at[gidx_v.at[i]], obuf[p],
                              sem_g[p]).wait()

    def issue_write(i, p):
        base = pl.multiple_of(base_w + i * BLK, BLK)
        pltpu.async_copy(obuf[p], out_hbm.at[pl.ds(base, BLK), pl.ds(0, D)],
                         sem_w[p])

    def wait_write(p):
        pltpu.make_async_copy(obuf[p],
                              out_hbm.at[pl.ds(base_w, BLK), pl.ds(0, D)],
                              sem_w[p]).wait()

    for i in range(LAG):
        issue_gathers(i, i)

    def step(i, p):
        pb = (p + LAG) % RING

        @pl.when(i + LAG < NCHUNK)
        def _():
            @pl.when(i + LAG >= RING)
            def _():
                wait_write(pb)
            issue_gathers(i + LAG, pb)

        wait_gathers(i, p)
        issue_write(i, p)

    def group(j, carry):
        for u in range(RING):
            step(RING * j + u, u)
        return carry

    lax.fori_loop(0, NCHUNK // RING, group, 0)
    for w in range(NCHUNK - RING + LAG, NCHUNK):
        wait_write(w % RING)

    c0 = 2 * wid
    pltpu.sync_copy(cidx_hbm.at[pl.ds(pl.multiple_of(c0 * 4, 8), 8)], ci_v)
    pltpu.async_copy(cond_t.at[ci_v], cbuf, sem_g[0]).wait()
    for cell in range(2):
        row = (c0 + cell) * L
        for j in range(D // 16):
            p = j * 16
            fbuf[0, pl.ds(p, 16)] = cbuf[4 * cell + p // 64, pl.ds(p % 64, 16)]
        pltpu.sync_copy(fbuf, out_hbm.at[pl.ds(row, 1), pl.ds(0, D)])


@functools.partial(jax.jit, static_argnames=())
def _sc_assemble(gidx, cidx, gene_table, cond_table):
    mesh = plsc.VectorSubcoreMesh(core_axis_name="c", subcore_axis_name="s")
    return pl.kernel(
        _sc_body,
        out_type=jax.ShapeDtypeStruct((N, TWO_D), jnp.float32),
        mesh=mesh,
        scratch_types=(
            [pltpu.VMEM((NCHUNK, BLK), jnp.int32)]
            + [pltpu.VMEM((BLK, D), jnp.float32)] * RING
            + [
                pltpu.VMEM((8,), jnp.int32),
                pltpu.VMEM((8, 128), jnp.float32),
                pltpu.VMEM((1, D), jnp.float32),
            ]
            + [pltpu.SemaphoreType.DMA] * (2 * RING)
        ),
    )(gidx, cidx, gene_table, cond_table)


def kernel(gene, expr, cond, batch, pad, gene_table, bin_embeddings,
           W1, b1, W2, b2, cond_table, batch_table):
    expr_table = _expr_table(bin_embeddings, W1, b1, W2, b2)

    zcol = jnp.zeros((C, 1), jnp.int32)
    gidx = jnp.concatenate([zcol, gene], axis=1).reshape(N // BLK, BLK)
    cidx = cond.reshape(-1)

    eidx2 = jnp.concatenate([zcol, expr], axis=1).reshape(N // TCBM, 1, TCBM)
    bidx_blk = jnp.pad(batch.reshape(N // TCBM, 1, TCBM // L),
                       ((0, 0), (0, 0), (0, 8 - TCBM // L)))

    cond_table_p = jnp.pad(cond_table, ((0, 0), (0, 64)))

    sc_out = _sc_assemble(gidx, cidx, gene_table, cond_table_p)
    out = _expr_half(sc_out, eidx2, bidx_blk, expr_table, batch_table)

    final_emb = out.reshape(C, L, TWO_D)
    key_padding_mask = jnp.concatenate(
        [jnp.zeros((C, 1), dtype=bool), pad.astype(bool)], axis=1)
    return (final_emb, key_padding_mask)


# speedup vs baseline: 4.0585x; 1.0083x over previous
import functools

import jax
import jax.numpy as jnp
from jax import lax
from jax.experimental import pallas as pl
from jax.experimental.pallas import tpu as pltpu
from jax.experimental.pallas import tpu_sc as plsc

C, L1, D = 64, 2047, 256
NUM_BINS, HID, BATCH_MAX = 32, 128, 256
L = L1 + 1
N = C * L
TWO_D = 2 * D

NC, NS = 2, 16
NW = NC * NS
ROWS_PER_W = N // NW
BLK = 128
NCHUNK = ROWS_PER_W // BLK
RING = 2
LAG = 1

TCBM = 8192
KTAB = NUM_BINS


def _expr_table_body(bins_ref, w1_ref, b1_ref, w2_ref, b2_ref, out_ref):
    nb, d = out_ref.shape
    bins = bins_ref[...]
    vals = lax.broadcasted_iota(jnp.int32, (nb, 1), 0).astype(jnp.float32)
    h = jnp.maximum(vals * w1_ref[...] + b1_ref[...], 0.0)
    enc = lax.dot_general(h, w2_ref[...], (((1,), (0,)), ((), ())),
                          precision=lax.Precision.DEFAULT) + b2_ref[...]
    sim = lax.dot_general(enc, bins, (((1,), (1,)), ((), ())),
                          precision=lax.Precision.DEFAULT)
    col = lax.broadcasted_iota(jnp.int32, (nb, nb), 1)
    sim = jnp.where(col == 0, -1e30, sim)
    m = jnp.max(sim, axis=-1, keepdims=True)
    p = jnp.exp(sim - m)
    p = p / jnp.sum(p, axis=-1, keepdims=True)
    emb = lax.dot_general(p, bins, (((1,), (0,)), ((), ())),
                          precision=lax.Precision.DEFAULT)
    row = lax.broadcasted_iota(jnp.int32, (nb, d), 0)
    out_ref[...] = jnp.where(row == 0, bins[0:1, :], emb)


def _expr_table(bin_embeddings, w1, b1, w2, b2):
    return pl.pallas_call(
        _expr_table_body,
        out_shape=jax.ShapeDtypeStruct((NUM_BINS, D), jnp.float32),
    )(bin_embeddings, w1.reshape(1, HID), b1.reshape(1, HID),
      w2, b2.reshape(1, D))


def _onehot_rows(e, ct, nrow, ktab):
    k = lax.broadcasted_iota(jnp.int32, (1, ktab), 1)
    oh = (e.reshape(nrow, 1) == k).astype(jnp.bfloat16)
    t1 = ct.astype(jnp.bfloat16)
    r1 = ct - t1.astype(jnp.float32)
    t2 = r1.astype(jnp.bfloat16)
    t3 = (r1 - t2.astype(jnp.float32)).astype(jnp.bfloat16)
    dims = (((1,), (0,)), ((), ()))
    acc = lax.dot_general(oh, t1, dims, preferred_element_type=jnp.float32)
    acc = acc + lax.dot_general(oh, t2, dims,
                                preferred_element_type=jnp.float32)
    acc = acc + lax.dot_general(oh, t3, dims,
                                preferred_element_type=jnp.float32)
    return acc


def _expr_half_body(out_any, eidx_ref, bidx_ref, ct_ref, bt_ref, out_ref):
    del out_any
    out_ref[...] = _onehot_rows(eidx_ref[0, 0, :], ct_ref[...], TCBM, KTAB)

    for cell in range(TCBM // L):
        b = bidx_ref[0, 0, cell]
        m = lax.broadcasted_iota(jnp.int32, (BATCH_MAX, 1), 0) == b
        row = jnp.sum(jnp.where(m, bt_ref[...], 0.0), axis=0, keepdims=True)
        out_ref[pl.ds(cell * L, 1), :] = row


def _expr_half(sc_out, eidx2, bidx_blk, ctable, batch_table):
    return pl.pallas_call(
        _expr_half_body,
        grid=(N // TCBM,),
        in_specs=[
            pl.BlockSpec(memory_space=pl.ANY),
            pl.BlockSpec((1, 1, TCBM), lambda j: (j, 0, 0)),
            pl.BlockSpec((1, 1, 8), lambda j: (j, 0, 0)),
            pl.BlockSpec((KTAB, D), lambda j: (0, 0)),
            pl.BlockSpec((BATCH_MAX, D), lambda j: (0, 0)),
        ],
        out_specs=pl.BlockSpec((TCBM, D), lambda j: (j, 1)),
        out_shape=jax.ShapeDtypeStruct((N, TWO_D), jnp.float32),
        input_output_aliases={0: 0},
    )(sc_out, eidx2, bidx_blk, ctable, batch_table)


def _sc_body(gidx_hbm, cidx_hbm, gene_t, cond_t, out_hbm, *scr):
    gidx_v = scr[0]
    obuf = scr[1:1 + RING]
    ci_v, cbuf, fbuf = scr[1 + RING:4 + RING]
    sem_g = scr[4 + RING:4 + 2 * RING]
    sem_w = scr[4 + 2 * RING:4 + 3 * RING]

    wid = lax.axis_index("s") * NC + lax.axis_index("c")
    base_w = pl.multiple_of(wid * ROWS_PER_W, ROWS_PER_W)
    chunk0 = pl.multiple_of(wid * NCHUNK, NCHUNK)

    pltpu.sync_copy(gidx_hbm.at[pl.ds(chunk0, NCHUNK)], gidx_v)

    def issue_gathers(i, p):
        pltpu.async_copy(gene_t.at[gidx_v.at[i]], obuf[p], sem_g[p])

    def wait_gathers(i, p):
        pltpu.make_async_copy(gene_t.at[gidx_v.at[i]], obuf[p],
                              sem_g[p]).wait()

    def issue_write(i, p):
        base = pl.multiple_of(base_w + i * BLK, BLK)
        pltpu.async_copy(obuf[p], out_hbm.at[pl.ds(base, BLK), pl.ds(0, D)],
                         sem_w[p])

    def wait_write(p):
        pltpu.make_async_copy(obuf[p],
                              out_hbm.at[pl.ds(base_w, BLK), pl.ds(0, D)],
                              sem_w[p]).wait()

    for i in range(LAG):
        issue_gathers(i, i)

    def step(i, p):
        pb = (p + LAG) % RING

        @pl.when(i + LAG < NCHUNK)
        def _():
            @pl.when(i + LAG >= RING)
            def _():
                wait_write(pb)
            issue_gathers(i + LAG, pb)

        wait_gathers(i, p)
        issue_write(i, p)

    def group(j, carry):
        for u in range(RING):
            step(RING * j + u, u)
        return carry

    lax.fori_loop(0, NCHUNK // RING, group, 0)
    for w in range(NCHUNK - RING + LAG, NCHUNK):
        wait_write(w % RING)

    c0 = 2 * wid
    pltpu.sync_copy(cidx_hbm.at[pl.ds(pl.multiple_of(c0 * 4, 8), 8)], ci_v)
    pltpu.async_copy(cond_t.at[ci_v], cbuf, sem_g[0]).wait()
    for cell in range(2):
        row = (c0 + cell) * L
        for j in range(D // 16):
            p = j * 16
            fbuf[0, pl.ds(p, 16)] = cbuf[4 * cell + p // 64, pl.ds(p % 64, 16)]
        pltpu.sync_copy(fbuf, out_hbm.at[pl.ds(row, 1), pl.ds(0, D)])


@functools.partial(jax.jit, static_argnames=())
def _sc_assemble(gidx, cidx, gene_table, cond_table):
    mesh = plsc.VectorSubcoreMesh(core_axis_name="c", subcore_axis_name="s")
    return pl.kernel(
        _sc_body,
        out_type=jax.ShapeDtypeStruct((N, TWO_D), jnp.float32),
        mesh=mesh,
        scratch_types=(
            [pltpu.VMEM((NCHUNK, BLK), jnp.int32)]
            + [pltpu.VMEM((BLK, D), jnp.float32)] * RING
            + [
                pltpu.VMEM((8,), jnp.int32),
                pltpu.VMEM((8, 128), jnp.float32),
                pltpu.VMEM((1, D), jnp.float32),
            ]
            + [pltpu.SemaphoreType.DMA] * (2 * RING)
        ),
    )(gidx, cidx, gene_table, cond_table)


def kernel(gene, expr, cond, batch, pad, gene_table, bin_embeddings,
           W1, b1, W2, b2, cond_table, batch_table):
    expr_table = _expr_table(bin_embeddings, W1, b1, W2, b2)

    zcol = jnp.zeros((C, 1), jnp.int32)
    gidx = jnp.concatenate([zcol, gene], axis=1).reshape(N // BLK, BLK)
    cidx = cond.reshape(-1)

    eidx2 = jnp.concatenate([zcol, expr], axis=1).reshape(N // TCBM, 1, TCBM)
    bidx_blk = jnp.pad(batch.reshape(N // TCBM, 1, TCBM // L),
                       ((0, 0), (0, 0), (0, 8 - TCBM // L)))

    cond_table_p = jnp.pad(cond_table, ((0, 0), (0, 64)))

    sc_out = _sc_assemble(gidx, cidx, gene_table, cond_table_p)
    out = _expr_half(sc_out, eidx2, bidx_blk, expr_table, batch_table)

    final_emb = out.reshape(C, L, TWO_D)
    key_padding_mask = jnp.concatenate(
        [jnp.zeros((C, 1), dtype=bool), pad.astype(bool)], axis=1)
    return (final_emb, key_padding_mask)
